# baseline (speedup 1.0000x reference)
import functools

import jax
import jax.numpy as jnp
from jax import lax
from jax.experimental import pallas as pl
from jax.experimental.pallas import tpu as pltpu
from jax.experimental.pallas import tpu_sc as plsc

B = 16
TOTAL = 32768
IN = 128
H = 128
DIST = 256
ROWS = 4096
SPLIT = 8192

_info = plsc.get_sparse_core_info()
_NC, _NS = _info.num_cores, _info.num_subcores
_NW = _NC * _NS
_RW = TOTAL // _NW

_sc_mesh = plsc.VectorSubcoreMesh(core_axis_name="c", subcore_axis_name="s")

_LANES = 16


@functools.partial(
    pl.kernel, mesh=_sc_mesh,
    out_type=jax.ShapeDtypeStruct((B * TOTAL,), jnp.float32),
    scratch_types=[
        pltpu.VMEM((B,), jnp.int32),
        pltpu.VMEM((B,), jnp.int32),
        pltpu.VMEM((B * _RW,), jnp.float32),
        pltpu.SemaphoreType.DMA,
    ],
)
def _sc_segment_onehot(starts_hbm, ends_hbm, oh_hbm, sv, ev, oh_v, sem):
    wid = lax.axis_index("s") * _NC + lax.axis_index("c")
    base = wid * _RW
    pltpu.sync_copy(starts_hbm, sv)
    pltpu.sync_copy(ends_hbm, ev)
    lane = lax.iota(jnp.int32, _LANES)
    one = jnp.full((_LANES,), 1.0, jnp.float32)
    zero = jnp.zeros((_LANES,), jnp.float32)
    s_vec = sv[...]
    e_vec = ev[...]
    copies = []
    for b in range(B):
        s_b = jnp.full((_LANES,), s_vec[b], jnp.int32)
        e_b = jnp.full((_LANES,), e_vec[b], jnp.int32)

        def body(i, carry, s_b=s_b, e_b=e_b, b=b):
            r_vec = jnp.full((_LANES,), base + i * _LANES, jnp.int32) + lane
            m = jnp.logical_and(s_b <= r_vec, r_vec < e_b)
            oh_v[pl.ds(b * _RW + i * _LANES, _LANES)] = jnp.where(m, one, zero)
            return carry

        lax.fori_loop(0, _RW // _LANES, body, 0)
        copies.append(pltpu.async_copy(oh_v.at[pl.ds(b * _RW, _RW)],
                                       oh_hbm.at[pl.ds(b * TOTAL + base, _RW)],
                                       sem))
    for c in copies:
        c.wait()


def _dense_tail(tt_contrib, traj_ref, nbr_ref, dist_ref, ht_ref,
                w1_ref, w2_ref, w3_ref, be_ref, wih_ref, whh_ref,
                bih_ref, bhh_ref, out_ref):
    bf = jnp.bfloat16
    emb = jnp.dot(nbr_ref[...].astype(bf), w1_ref[...], preferred_element_type=jnp.float32)
    emb = emb + jnp.dot(dist_ref[...].astype(bf), w3_ref[...], preferred_element_type=jnp.float32)
    emb = emb + tt_contrib
    x = jnp.maximum(emb, 0.0)

    h = ht_ref[...]
    gi = jnp.dot(x.astype(bf), wih_ref[...], preferred_element_type=jnp.float32) + bih_ref[...]
    gh = jnp.dot(h.astype(bf), whh_ref[...], preferred_element_type=jnp.float32) + bhh_ref[...]
    r = jax.nn.sigmoid(gi[:, 0:H] + gh[:, 0:H])
    z = jax.nn.sigmoid(gi[:, H:2 * H] + gh[:, H:2 * H])
    n = jnp.tanh(gi[:, 2 * H:3 * H] + r * gh[:, 2 * H:3 * H])
    out_ref[...] = (1.0 - z) * n + z * h


def _proj_traj(traj_ref, w2_ref, be_ref):
    p = jnp.dot(traj_ref[...], w2_ref[...], preferred_element_type=jnp.float32)
    return p + be_ref[...]


def _step_inline(se_ref, traj_ref, nbr_ref, dist_ref, ht_ref,
                 w1_ref, w2_ref, w3_ref, be_ref, wih_ref, whh_ref,
                 bih_ref, bhh_ref, out_ref):
    rows = jax.lax.broadcasted_iota(jnp.int32, (ROWS, B), 0) + pl.program_id(0) * ROWS
    onehot = jnp.logical_and(se_ref[0:1, :] <= rows,
                             rows < se_ref[1:2, :]).astype(jnp.float32)
    p = _proj_traj(traj_ref, w2_ref, be_ref)
    tt = jnp.dot(onehot, p, preferred_element_type=jnp.float32)
    _dense_tail(tt, traj_ref, nbr_ref, dist_ref, ht_ref, w1_ref, w2_ref,
                w3_ref, be_ref, wih_ref, whh_ref, bih_ref, bhh_ref, out_ref)


def _step_sc(alias_ref, oh_ref, traj_ref, nbr_ref, dist_ref, ht_ref,
             w1_ref, w2_ref, w3_ref, be_ref, wih_ref, whh_ref,
             bih_ref, bhh_ref, out_ref):
    del alias_ref
    p = _proj_traj(traj_ref, w2_ref, be_ref)
    tt = lax.dot_general(oh_ref[...], p, (((0,), (0,)), ((), ())),
                         preferred_element_type=jnp.float32)
    _dense_tail(tt, traj_ref, nbr_ref, dist_ref, ht_ref, w1_ref, w2_ref,
                w3_ref, be_ref, wih_ref, whh_ref, bih_ref, bhh_ref, out_ref)


def kernel(traj_input, neighbor_t, dist, neighbors_idx_start, neighbors_idx_end,
           ht, W_emb, b_emb, w_ih, w_hh, b_ih, b_hh):
    starts = neighbors_idx_start.astype(jnp.int32)
    ends = neighbors_idx_end.astype(jnp.int32)
    onehot_t = _sc_segment_onehot(starts, ends).reshape(B, TOTAL)

    se = jnp.stack([starts, ends])
    w1 = W_emb[:, :IN].T.astype(jnp.bfloat16)
    w2 = W_emb[:, IN:IN + H].T
    w3 = W_emb[:, IN + H:].T.astype(jnp.bfloat16)
    be = b_emb.reshape(1, H)
    wih = w_ih.T.astype(jnp.bfloat16)
    whh = w_hh.T.astype(jnp.bfloat16)
    bih = b_ih.reshape(1, 3 * H)
    bhh = b_hh.reshape(1, 3 * H)

    rep = lambda i: (0, 0)
    weight_specs = [
        pl.BlockSpec((B, H), rep),
        pl.BlockSpec((ROWS, IN), lambda i: (i, 0)),
        pl.BlockSpec((ROWS, DIST), lambda i: (i, 0)),
        pl.BlockSpec((ROWS, H), lambda i: (i, 0)),
        pl.BlockSpec((IN, H), rep),
        pl.BlockSpec((H, H), rep),
        pl.BlockSpec((DIST, H), rep),
        pl.BlockSpec((1, H), rep),
        pl.BlockSpec((H, 3 * H), rep),
        pl.BlockSpec((H, 3 * H), rep),
        pl.BlockSpec((1, 3 * H), rep),
        pl.BlockSpec((1, 3 * H), rep),
    ]
    dense_args = (traj_input, neighbor_t, dist, ht,
                  w1, w2, w3, be, wih, whh, bih, bhh)
    out_sds = jax.ShapeDtypeStruct((TOTAL, H), jnp.float32)

    out_a = pl.pallas_call(
        _step_inline,
        grid=(SPLIT // ROWS,),
        in_specs=[pl.BlockSpec((2, B), rep)] + weight_specs,
        out_specs=pl.BlockSpec((ROWS, H), lambda i: (i, 0)),
        out_shape=out_sds,
        compiler_params=pltpu.CompilerParams(
            dimension_semantics=("parallel",)),
    )(se, *dense_args)

    off = SPLIT // ROWS
    out = pl.pallas_call(
        _step_sc,
        grid=((TOTAL - SPLIT) // ROWS,),
        in_specs=[pl.BlockSpec(memory_space=pl.ANY),
                  pl.BlockSpec((B, ROWS), lambda i, off=off: (0, i + off))]
                 + [pl.BlockSpec(s.block_shape,
                                 (lambda m: (lambda i, m=m: m(i + off)))(s.index_map))
                    for s in weight_specs],
        out_specs=pl.BlockSpec((ROWS, H), lambda i, off=off: (i + off, 0)),
        out_shape=out_sds,
        input_output_aliases={0: 0},
        compiler_params=pltpu.CompilerParams(
            dimension_semantics=("arbitrary",)),
    )(out_a, onehot_t, *dense_args)
    return out


# speedup vs baseline: 1.0772x; 1.0772x over previous
import functools

import jax
import jax.numpy as jnp
from jax import lax
from jax.experimental import pallas as pl
from jax.experimental.pallas import tpu as pltpu
from jax.experimental.pallas import tpu_sc as plsc

B = 16
TOTAL = 32768
IN = 128
H = 128
DIST = 256
ROWS = 4096

_info = plsc.get_sparse_core_info()
_NC, _NS = _info.num_cores, _info.num_subcores
_NW = _NC * _NS
_RW = TOTAL // _NW

_sc_mesh = plsc.VectorSubcoreMesh(core_axis_name="c", subcore_axis_name="s")

_LANES = 16


@functools.partial(
    pl.kernel, mesh=_sc_mesh,
    out_type=jax.ShapeDtypeStruct((B * TOTAL,), jnp.float32),
    scratch_types=[
        pltpu.VMEM((B,), jnp.int32),
        pltpu.VMEM((B,), jnp.int32),
        pltpu.VMEM((B * _RW,), jnp.float32),
        pltpu.SemaphoreType.DMA,
    ],
)
def _sc_segment_onehot(starts_hbm, ends_hbm, oh_hbm, sv, ev, oh_v, sem):
    wid = lax.axis_index("s") * _NC + lax.axis_index("c")
    base = wid * _RW
    pltpu.sync_copy(starts_hbm, sv)
    pltpu.sync_copy(ends_hbm, ev)
    lane = lax.iota(jnp.int32, _LANES)
    one = jnp.full((_LANES,), 1.0, jnp.float32)
    zero = jnp.zeros((_LANES,), jnp.float32)
    s_vec = sv[...]
    e_vec = ev[...]
    copies = []
    for b in range(B):
        s_b = jnp.full((_LANES,), s_vec[b], jnp.int32)
        e_b = jnp.full((_LANES,), e_vec[b], jnp.int32)

        def body(i, carry, s_b=s_b, e_b=e_b, b=b):
            r_vec = jnp.full((_LANES,), base + i * _LANES, jnp.int32) + lane
            m = jnp.logical_and(s_b <= r_vec, r_vec < e_b)
            oh_v[pl.ds(b * _RW + i * _LANES, _LANES)] = jnp.where(m, one, zero)
            return carry

        lax.fori_loop(0, _RW // _LANES, body, 0)
        copies.append(pltpu.async_copy(oh_v.at[pl.ds(b * _RW, _RW)],
                                       oh_hbm.at[pl.ds(b * TOTAL + base, _RW)],
                                       sem))
    for c in copies:
        c.wait()


def _fused_step(oh_ref, traj_ref, nbr_ref, dist_ref, ht_ref,
                w1_ref, w2_ref, w3_ref, be_ref, wih_ref, whh_ref,
                bih_ref, bhh_ref, out_ref):
    oh_t = oh_ref[...]

    p = jnp.dot(traj_ref[...], w2_ref[...], preferred_element_type=jnp.float32)
    p = p + be_ref[...]

    bf = jnp.bfloat16
    emb = jnp.dot(nbr_ref[...].astype(bf), w1_ref[...], preferred_element_type=jnp.float32)
    emb = emb + jnp.dot(dist_ref[...].astype(bf), w3_ref[...], preferred_element_type=jnp.float32)
    emb = emb + lax.dot_general(oh_t, p, (((0,), (0,)), ((), ())),
                                preferred_element_type=jnp.float32)
    x = jnp.maximum(emb, 0.0)

    h = ht_ref[...]
    gi = jnp.dot(x.astype(bf), wih_ref[...], preferred_element_type=jnp.float32) + bih_ref[...]
    gh = jnp.dot(h.astype(bf), whh_ref[...], preferred_element_type=jnp.float32) + bhh_ref[...]
    r = jax.nn.sigmoid(gi[:, 0:H] + gh[:, 0:H])
    z = jax.nn.sigmoid(gi[:, H:2 * H] + gh[:, H:2 * H])
    n = jnp.tanh(gi[:, 2 * H:3 * H] + r * gh[:, 2 * H:3 * H])
    out_ref[...] = (1.0 - z) * n + z * h


def kernel(traj_input, neighbor_t, dist, neighbors_idx_start, neighbors_idx_end,
           ht, W_emb, b_emb, w_ih, w_hh, b_ih, b_hh):
    onehot_t = _sc_segment_onehot(
        neighbors_idx_start.astype(jnp.int32),
        neighbors_idx_end.astype(jnp.int32),
    ).reshape(B, TOTAL)

    w1 = W_emb[:, :IN].T.astype(jnp.bfloat16)
    w2 = W_emb[:, IN:IN + H].T
    w3 = W_emb[:, IN + H:].T.astype(jnp.bfloat16)
    be = b_emb.reshape(1, H)
    wih = w_ih.T.astype(jnp.bfloat16)
    whh = w_hh.T.astype(jnp.bfloat16)
    bih = b_ih.reshape(1, 3 * H)
    bhh = b_hh.reshape(1, 3 * H)

    grid = TOTAL // ROWS
    rep = lambda i: (0, 0)
    out = pl.pallas_call(
        _fused_step,
        grid=(grid,),
        in_specs=[
            pl.BlockSpec((B, ROWS), lambda i: (0, i)),
            pl.BlockSpec((B, H), rep),
            pl.BlockSpec((ROWS, IN), lambda i: (i, 0)),
            pl.BlockSpec((ROWS, DIST), lambda i: (i, 0)),
            pl.BlockSpec((ROWS, H), lambda i: (i, 0)),
            pl.BlockSpec((IN, H), rep),
            pl.BlockSpec((H, H), rep),
            pl.BlockSpec((DIST, H), rep),
            pl.BlockSpec((1, H), rep),
            pl.BlockSpec((H, 3 * H), rep),
            pl.BlockSpec((H, 3 * H), rep),
            pl.BlockSpec((1, 3 * H), rep),
            pl.BlockSpec((1, 3 * H), rep),
        ],
        out_specs=pl.BlockSpec((ROWS, H), lambda i: (i, 0)),
        out_shape=jax.ShapeDtypeStruct((TOTAL, H), jnp.float32),
        compiler_params=pltpu.CompilerParams(
            dimension_semantics=("parallel",)),
    )(onehot_t, traj_input, neighbor_t, dist, ht, w1, w2, w3, be, wih, whh, bih, bhh)
    return out


# speedup vs baseline: 1.1129x; 1.0331x over previous
import functools

import jax
import jax.numpy as jnp
from jax import lax
from jax.experimental import pallas as pl
from jax.experimental.pallas import tpu as pltpu
from jax.experimental.pallas import tpu_sc as plsc

B = 16
TOTAL = 32768
IN = 128
H = 128
DIST = 256
ROWS = 4096

_info = plsc.get_sparse_core_info()
_NC, _NS = _info.num_cores, _info.num_subcores
_NW = _NC * _NS
_RW = TOTAL // _NW

_sc_mesh = plsc.VectorSubcoreMesh(core_axis_name="c", subcore_axis_name="s",
                                  num_cores=1)

_LANES = 16


@functools.partial(
    pl.kernel, mesh=_sc_mesh,
    out_type=jax.ShapeDtypeStruct((B * TOTAL,), jnp.float32),
    scratch_types=[
        pltpu.VMEM((B,), jnp.int32),
        pltpu.VMEM((B,), jnp.int32),
        pltpu.VMEM((B * _RW,), jnp.float32),
        pltpu.SemaphoreType.DMA,
    ],
)
def _sc_segment_onehot(starts_hbm, ends_hbm, oh_hbm, sv, ev, oh_v, sem):
    wid = lax.axis_index("s") * _NC + lax.axis_index("c")
    base = wid * _RW
    pltpu.sync_copy(starts_hbm, sv)
    pltpu.sync_copy(ends_hbm, ev)
    lane = lax.iota(jnp.int32, _LANES)
    one = jnp.full((_LANES,), 1.0, jnp.float32)
    zero = jnp.zeros((_LANES,), jnp.float32)
    s_vec = sv[...]
    e_vec = ev[...]
    copies = []
    for b in range(B):
        s_b = jnp.full((_LANES,), s_vec[b], jnp.int32)
        e_b = jnp.full((_LANES,), e_vec[b], jnp.int32)

        def body(i, carry, s_b=s_b, e_b=e_b, b=b):
            r_vec = jnp.full((_LANES,), base + i * _LANES, jnp.int32) + lane
            m = jnp.logical_and(s_b <= r_vec, r_vec < e_b)
            oh_v[pl.ds(b * _RW + i * _LANES, _LANES)] = jnp.where(m, one, zero)
            return carry

        lax.fori_loop(0, _RW // _LANES, body, 0)
        copies.append(pltpu.async_copy(oh_v.at[pl.ds(b * _RW, _RW)],
                                       oh_hbm.at[pl.ds(b * TOTAL + base, _RW)],
                                       sem))
    for c in copies:
        c.wait()


def _fused_step(oh_ref, traj_ref, nbr_ref, dist_ref, ht_ref,
                w1_ref, w2_ref, w3_ref, be_ref, wih_ref, whh_ref,
                bih_ref, bhh_ref, out_ref):
    oh_t = oh_ref[...]

    p = jnp.dot(traj_ref[...], w2_ref[...], preferred_element_type=jnp.float32)
    p = p + be_ref[...]

    bf = jnp.bfloat16
    emb = jnp.dot(nbr_ref[...].astype(bf), w1_ref[...], preferred_element_type=jnp.float32)
    emb = emb + jnp.dot(dist_ref[...].astype(bf), w3_ref[...], preferred_element_type=jnp.float32)
    emb = emb + lax.dot_general(oh_t, p, (((0,), (0,)), ((), ())),
                                preferred_element_type=jnp.float32)
    x = jnp.maximum(emb, 0.0)

    h = ht_ref[...]
    gi = jnp.dot(x.astype(bf), wih_ref[...], preferred_element_type=jnp.float32) + bih_ref[...]
    gh = jnp.dot(h.astype(bf), whh_ref[...], preferred_element_type=jnp.float32) + bhh_ref[...]
    r = jax.nn.sigmoid(gi[:, 0:H] + gh[:, 0:H])
    z = jax.nn.sigmoid(gi[:, H:2 * H] + gh[:, H:2 * H])
    n = jnp.tanh(gi[:, 2 * H:3 * H] + r * gh[:, 2 * H:3 * H])
    out_ref[...] = (1.0 - z) * n + z * h


def kernel(traj_input, neighbor_t, dist, neighbors_idx_start, neighbors_idx_end,
           ht, W_emb, b_emb, w_ih, w_hh, b_ih, b_hh):
    onehot_t = _sc_segment_onehot(
        neighbors_idx_start.astype(jnp.int32),
        neighbors_idx_end.astype(jnp.int32),
    ).reshape(B, TOTAL)

    w1 = W_emb[:, :IN].T.astype(jnp.bfloat16)
    w2 = W_emb[:, IN:IN + H].T
    w3 = W_emb[:, IN + H:].T.astype(jnp.bfloat16)
    be = b_emb.reshape(1, H)
    wih = w_ih.T.astype(jnp.bfloat16)
    whh = w_hh.T.astype(jnp.bfloat16)
    bih = b_ih.reshape(1, 3 * H)
    bhh = b_hh.reshape(1, 3 * H)

    grid = TOTAL // ROWS
    rep = lambda i: (0, 0)
    out = pl.pallas_call(
        _fused_step,
        grid=(grid,),
        in_specs=[
            pl.BlockSpec((B, ROWS), lambda i: (0, i)),
            pl.BlockSpec((B, H), rep),
            pl.BlockSpec((ROWS, IN), lambda i: (i, 0)),
            pl.BlockSpec((ROWS, DIST), lambda i: (i, 0)),
            pl.BlockSpec((ROWS, H), lambda i: (i, 0)),
            pl.BlockSpec((IN, H), rep),
            pl.BlockSpec((H, H), rep),
            pl.BlockSpec((DIST, H), rep),
            pl.BlockSpec((1, H), rep),
            pl.BlockSpec((H, 3 * H), rep),
            pl.BlockSpec((H, 3 * H), rep),
            pl.BlockSpec((1, 3 * H), rep),
            pl.BlockSpec((1, 3 * H), rep),
        ],
        out_specs=pl.BlockSpec((ROWS, H), lambda i: (i, 0)),
        out_shape=jax.ShapeDtypeStruct((TOTAL, H), jnp.float32),
        compiler_params=pltpu.CompilerParams(
            dimension_semantics=("parallel",)),
    )(onehot_t, traj_input, neighbor_t, dist, ht, w1, w2, w3, be, wih, whh, bih, bhh)
    return out
